# Initial kernel scaffold; baseline (speedup 1.0000x reference)
import jax
import jax.numpy as jnp
from jax.experimental import pallas as pl


def kernel(flat_features, offsets, weight, bias):
    raise NotImplementedError("write your pallas kernel here")


# speedup vs baseline: 221.5582x; 221.5582x over previous
import functools

import jax
import jax.numpy as jnp
from jax import lax
from jax.experimental import pallas as pl
from jax.experimental.pallas import tpu as pltpu
from jax.experimental.pallas import tpu_sc as plsc

V = 1000000
D = 128
N = 819200
B = 16384

NC = 2
NS = 16
NW = NC * NS
BAGS_W = B // NW
BAGS_SC = B // NC
C = 128
TRASH = BAGS_SC


def _sc_embedding_bag(ff, off_ext, weight, bias_rows):
    mesh = plsc.VectorSubcoreMesh(core_axis_name="c", subcore_axis_name="s")

    @functools.partial(
        pl.kernel,
        out_type=jax.ShapeDtypeStruct((B, D), jnp.float32),
        mesh=mesh,
        compiler_params=pltpu.CompilerParams(needs_layout_passes=False),
        scratch_types=[
            pltpu.VMEM((520,), jnp.int32),
            pltpu.VMEM((1, C), jnp.int32),
            pltpu.VMEM((1, C), jnp.int32),
            pltpu.VMEM((C, D), jnp.float32),
            pltpu.VMEM_SHARED((BAGS_SC + 8, D), jnp.float32),
            pltpu.SemaphoreType.DMA,
        ],
    )
    def k(ff_hbm, off_hbm, w_hbm, binit_hbm, out_hbm,
          off_v, fidx, dst, rows, spacc, sem):
        c = lax.axis_index("c")
        s = lax.axis_index("s")
        w = c * NS + s

        pltpu.sync_copy(off_hbm.at[pl.ds(w * BAGS_W, 520)], off_v)
        pltpu.sync_copy(binit_hbm, spacc.at[pl.ds(s * BAGS_W, BAGS_W)])

        start = off_v[pl.ds(0, 16)][0]
        end = off_v[pl.ds(BAGS_W - 8, 16)][8]
        k0 = start >> 7
        k1 = (end + (C - 1)) >> 7
        lanes = lax.iota(jnp.int32, 16)

        def body(kk, carry):
            base = kk * C
            pltpu.sync_copy(ff_hbm.at[pl.ds(base, C)], fidx.at[0])
            cp = pltpu.async_copy(w_hbm.at[fidx.at[0]], rows, sem)
            for j in range(C // 16):
                pos = base + j * 16 + lanes
                lo = jnp.zeros((16,), jnp.int32)
                hi = jnp.full((16,), BAGS_W + 1, jnp.int32)
                for _ in range(10):
                    mid = (lo + hi) >> 1
                    vo = plsc.load_gather(off_v, [mid])
                    le = vo <= pos
                    lo = jnp.where(le, mid + 1, lo)
                    hi = jnp.where(le, hi, mid)
                valid = (pos >= start) & (pos < end)
                d_ = jnp.where(valid, s * BAGS_W + (lo - 1), TRASH)
                dst[0, pl.ds(j * 16, 16)] = d_
            cp.wait()
            pltpu.sync_copy(rows, spacc.at[dst.at[0]], add=True)
            return carry

        lax.fori_loop(k0, k1, body, 0)

        pltpu.sync_copy(spacc.at[pl.ds(s * BAGS_W, BAGS_W)],
                        out_hbm.at[pl.ds(w * BAGS_W, BAGS_W)])

    return k(ff, off_ext, weight, bias_rows)


def kernel(flat_features, offsets, weight, bias):
    off_ext = jnp.concatenate(
        [offsets.astype(jnp.int32), jnp.full((8,), N, jnp.int32)])
    bias_rows = jnp.tile(bias.astype(jnp.float32)[None, :], (BAGS_W, 1))
    return _sc_embedding_bag(flat_features, off_ext, weight, bias_rows)
